# Initial kernel scaffold; baseline (speedup 1.0000x reference)
import jax
import jax.numpy as jnp
from jax.experimental import pallas as pl


def kernel(data, indices, updates):
    raise NotImplementedError("write your pallas kernel here")


# speedup vs baseline: 2.9955x; 2.9955x over previous
import functools

import jax
import jax.numpy as jnp
from jax import lax
from jax.experimental import pallas as pl
from jax.experimental.pallas import tpu as pltpu
from jax.experimental.pallas import tpu_sc as plsc

_L = 16
_NW = 32
_CHUNK = 128


def _scatter_nd_sc(data_ref, indices, updates):
  n_rows, d = data_ref.shape
  b = indices.shape[0]
  r_per_w = -(-n_rows // _NW)
  n_vregs = b // _L
  sel_cap = b + 2 * _L
  fin_cap = b + _L
  mesh = plsc.VectorSubcoreMesh(core_axis_name="c", subcore_axis_name="s")

  @functools.partial(
      pl.kernel,
      out_type=(),
      mesh=mesh,
      compiler_params=pltpu.CompilerParams(
          use_tc_tiling_on_sc=False, needs_layout_passes=False
      ),
      scratch_types=dict(
          idx_v=pltpu.VMEM((b,), jnp.int32),
          sel_d=pltpu.VMEM((sel_cap,), jnp.int32),
          sel_p=pltpu.VMEM((sel_cap,), jnp.int32),
          fin_d=pltpu.VMEM((fin_cap,), jnp.int32),
          fin_p=pltpu.VMEM((fin_cap,), jnp.int32),
          win=pltpu.VMEM((r_per_w,), jnp.int32),
          dchunk=pltpu.VMEM((_CHUNK,), jnp.int32),
          pchunk=pltpu.VMEM((_CHUNK,), jnp.int32),
          rows=pltpu.VMEM((_CHUNK, d), jnp.float32),
          sem=pltpu.SemaphoreType.DMA,
      ),
  )
  def scatter_kernel(data_hbm, idx_hbm, upd_hbm, *, idx_v, sel_d, sel_p,
                     fin_d, fin_p, win, dchunk, pchunk, rows, sem):
    wid = lax.axis_index("s") * 2 + lax.axis_index("c")
    lo = wid * r_per_w
    lanes = lax.iota(jnp.int32, _L)
    lo_v = jnp.broadcast_to(lo, (_L,))
    hi_v = jnp.broadcast_to(jnp.minimum(lo + r_per_w, n_rows), (_L,))

    pltpu.sync_copy(idx_hbm, idx_v)

    def select_body(i, n):
      v = idx_v[pl.ds(i * _L, _L)]
      m = (v >= lo_v) & (v < hi_v)
      cum = plsc.cumsum(m.astype(jnp.int32))
      off = jnp.broadcast_to(n, (_L,)) + cum - 1
      plsc.store_scatter(sel_d, [off], v, mask=m)
      plsc.store_scatter(sel_p, [off], i * _L + lanes, mask=m)
      return n + jnp.max(cum)

    n = lax.fori_loop(0, n_vregs, select_body, jnp.int32(0))
    n_v = jnp.broadcast_to(n, (_L,))

    n_ch = (n + _L - 1) // _L

    def winner_body(j, _):
      base = j * _L
      k = base + lanes
      d = sel_d[pl.ds(base, _L)]
      keep = k < n_v
      for s in range(1, _L):
        sh = sel_d[pl.ds(base + s, _L)]
        keep = keep & ~((sh == d) & (k + s < n_v))
      plsc.store_scatter(win, [d - lo_v], k, mask=keep)
      return 0

    lax.fori_loop(0, n_ch, winner_body, 0)

    def keep_body(j, m2):
      base = j * _L
      k = base + lanes
      valid = k < n_v
      dv = sel_d[pl.ds(base, _L)]
      pv = sel_p[pl.ds(base, _L)]
      w = plsc.load_gather(win, [dv - lo_v], mask=valid)
      keep = valid & (w == k)
      cum = plsc.cumsum(keep.astype(jnp.int32))
      off = jnp.broadcast_to(m2, (_L,)) + cum - 1
      plsc.store_scatter(fin_d, [off], dv, mask=keep)
      plsc.store_scatter(fin_p, [off], pv, mask=keep)
      return m2 + jnp.max(cum)

    m2 = lax.fori_loop(0, n_ch, keep_body, jnp.int32(0))

    last = jnp.broadcast_to(jnp.maximum(m2 - 1, 0), (_L,))
    d_last = plsc.load_gather(fin_d, [last])
    p_last = plsc.load_gather(fin_p, [last])
    m2r = ((m2 + _CHUNK - 1) // _CHUNK) * _CHUNK
    a0 = (m2 // _L) * _L
    m2_v = jnp.broadcast_to(m2, (_L,))

    def pad_body(t, _):
      a = a0 + t * _L
      pos = a + lanes
      dv = fin_d[pl.ds(a, _L)]
      pv = fin_p[pl.ds(a, _L)]
      fin_d[pl.ds(a, _L)] = jnp.where(pos < m2_v, dv, d_last)
      fin_p[pl.ds(a, _L)] = jnp.where(pos < m2_v, pv, p_last)
      return 0

    lax.fori_loop(0, (m2r - a0) // _L, pad_body, 0)

    def scatter_body(c, _):
      for t in range(_CHUNK // _L):
        dchunk[pl.ds(t * _L, _L)] = fin_d[pl.ds(c * _CHUNK + t * _L, _L)]
        pchunk[pl.ds(t * _L, _L)] = fin_p[pl.ds(c * _CHUNK + t * _L, _L)]
      pltpu.async_copy(upd_hbm.at[pchunk], rows, sem).wait()
      pltpu.async_copy(rows, data_hbm.at[dchunk], sem).wait()
      return 0

    lax.fori_loop(0, m2r // _CHUNK, scatter_body, 0)

  scatter_kernel(data_ref, indices, updates)


def kernel(data, indices, updates):
  idx_flat = indices.reshape((indices.shape[0],))
  out_ref = jax.new_ref(data)
  _scatter_nd_sc(out_ref, idx_flat, updates)
  return out_ref[...]
